# baseline (speedup 1.0000x reference)
import functools

import jax
import jax.numpy as jnp
from jax import lax
from jax.experimental import pallas as pl
from jax.experimental.pallas import tpu as pltpu
from jax.experimental.pallas import tpu_sc as plsc

D = 32
CHUNK = 128


def kernel(indices, Z):
    B = indices.shape[0]
    info = plsc.get_sparse_core_info()
    NC, NS = info.num_cores, info.num_subcores
    NW = NC * NS
    b_per_w = B // NW
    n_chunks = b_per_w // CHUNK
    mesh = plsc.VectorSubcoreMesh(core_axis_name="c", subcore_axis_name="s")

    idx2d = indices.reshape(B // CHUNK, CHUNK)

    @functools.partial(
        pl.kernel,
        mesh=mesh,
        compiler_params=pltpu.CompilerParams(use_tc_tiling_on_sc=False),
        out_type=jax.ShapeDtypeStruct((B, D), jnp.float32),
        scratch_types=[
            pltpu.VMEM((n_chunks, CHUNK), jnp.int32),
            pltpu.VMEM((n_chunks, CHUNK, D), jnp.float32),
            pltpu.SemaphoreType.DMA,
        ],
    )
    def gather_kernel(idx_hbm, table_hbm, out_hbm, idx_v, rows_v, sem):
        wid = lax.axis_index("s") * NC + lax.axis_index("c")
        base_chunk = wid * n_chunks
        pltpu.sync_copy(idx_hbm.at[pl.ds(base_chunk, n_chunks)], idx_v)
        copies = [
            pltpu.async_copy(table_hbm.at[idx_v.at[j]], rows_v.at[j], sem)
            for j in range(n_chunks)
        ]
        for c in copies:
            c.wait()
        for j in range(n_chunks):
            pltpu.sync_copy(
                rows_v.at[j],
                out_hbm.at[pl.ds((base_chunk + j) * CHUNK, CHUNK)],
            )

    return gather_kernel(idx2d, Z)


# speedup vs baseline: 7.2972x; 7.2972x over previous
import functools

import jax
import jax.numpy as jnp
from jax import lax
from jax.experimental import pallas as pl
from jax.experimental.pallas import tpu as pltpu
from jax.experimental.pallas import tpu_sc as plsc

D = 32


def kernel(indices, Z):
    B = indices.shape[0]
    info = plsc.get_sparse_core_info()
    NC, NS = info.num_cores, info.num_subcores
    NW = NC * NS
    BW = B // NW

    mesh = plsc.VectorSubcoreMesh(core_axis_name="c", subcore_axis_name="s")
    Zt = Z.T

    @functools.partial(
        pl.kernel,
        mesh=mesh,
        compiler_params=pltpu.CompilerParams(needs_layout_passes=False),
        out_type=jax.ShapeDtypeStruct((D, B), jnp.float32),
        scratch_types=[
            pltpu.VMEM((BW,), jnp.int32),
            pltpu.VMEM((2, 8, D, 128), jnp.float32),
            pltpu.VMEM((D, BW), jnp.float32),
            pltpu.SemaphoreType.DMA,
            pltpu.SemaphoreType.DMA,
        ],
    )
    def gather_kernel(idx_hbm, zt_hbm, out_hbm, idx_v, ring, stg, sem0, sem1):
        wid = lax.axis_index("s") * NC + lax.axis_index("c")
        base = pl.multiple_of(wid * BW, 128)
        pltpu.sync_copy(idx_hbm.at[pl.ds(base, BW)], idx_v)
        row0 = lax.broadcasted_iota(jnp.int32, (16,), 0)
        row1 = row0 + 16
        sems = [sem0, sem1]

        def issue(iv, h2, s):
            i0 = iv[8 * h2 + s]
            off = pl.multiple_of((i0 >> 7) << 7, 128)
            pltpu.async_copy(
                zt_hbm.at[:, pl.ds(off, 128)], ring.at[h2, s], sems[h2]
            )

        def drain(h2):
            for s in range(8):
                pltpu.make_async_copy(
                    zt_hbm.at[:, pl.ds(0, 128)], ring.at[h2, s], sems[h2]
                ).wait()

        def extract(iv, t, h2, s):
            i0 = iv[8 * h2 + s]
            cvec = jnp.broadcast_to(i0 & 127, (16,))
            pvec = jnp.broadcast_to(t + 8 * h2 + s, (16,))
            v0 = plsc.load_gather(ring.at[h2, s], [row0, cvec])
            v1 = plsc.load_gather(ring.at[h2, s], [row1, cvec])
            plsc.store_scatter(stg, [row0, pvec], v0)
            plsc.store_scatter(stg, [row1, pvec], v1)

        iv0 = idx_v[pl.ds(0, 16)]
        for h2 in range(2):
            for s in range(8):
                issue(iv0, h2, s)

        @pl.loop(0, BW - 16, step=16)
        def _body(t):
            iv = idx_v[pl.ds(t, 16)]
            ivn = idx_v[pl.ds(t + 16, 16)]
            for h2 in range(2):
                drain(h2)
                for s in range(8):
                    extract(iv, t, h2, s)
                for s in range(8):
                    issue(ivn, h2, s)

        ivt = idx_v[pl.ds(BW - 16, 16)]
        for h2 in range(2):
            drain(h2)
            for s in range(8):
                extract(ivt, BW - 16, h2, s)

        pltpu.sync_copy(stg, out_hbm.at[:, pl.ds(base, BW)])

    return gather_kernel(indices, Zt).T


# speedup vs baseline: 7.3089x; 1.0016x over previous
import functools

import jax
import jax.numpy as jnp
from jax import lax
from jax.experimental import pallas as pl
from jax.experimental.pallas import tpu as pltpu
from jax.experimental.pallas import tpu_sc as plsc

D = 32


def kernel(indices, Z):
    B = indices.shape[0]
    info = plsc.get_sparse_core_info()
    NC, NS = info.num_cores, info.num_subcores
    NW = NC * NS
    BW = B // NW

    mesh = plsc.VectorSubcoreMesh(core_axis_name="c", subcore_axis_name="s")
    Zt = Z.T

    @functools.partial(
        pl.kernel,
        mesh=mesh,
        compiler_params=pltpu.CompilerParams(needs_layout_passes=False),
        out_type=jax.ShapeDtypeStruct((D, B), jnp.float32),
        scratch_types=[
            pltpu.VMEM((BW,), jnp.int32),
            pltpu.VMEM((2, 12, D, 128), jnp.float32),
            pltpu.VMEM((D, BW), jnp.float32),
            pltpu.SemaphoreType.DMA,
            pltpu.SemaphoreType.DMA,
        ],
    )
    def gather_kernel(idx_hbm, zt_hbm, out_hbm, idx_v, ring, stg, sem0, sem1):
        wid = lax.axis_index("s") * NC + lax.axis_index("c")
        base = pl.multiple_of(wid * BW, 128)
        pltpu.sync_copy(idx_hbm.at[pl.ds(base, BW)], idx_v)
        row0 = lax.broadcasted_iota(jnp.int32, (16,), 0)
        row1 = row0 + 16
        sems = [sem0, sem1]

        def lane(iva, ivb, p):
            return iva[p] if p < 16 else ivb[p - 8]

        def issue(iva, ivb, h2, s):
            i0 = lane(iva, ivb, 12 * h2 + s)
            off = pl.multiple_of((i0 >> 7) << 7, 128)
            pltpu.async_copy(
                zt_hbm.at[:, pl.ds(off, 128)], ring.at[h2, s], sems[h2]
            )

        def drain(h2):
            for s in range(12):
                pltpu.make_async_copy(
                    zt_hbm.at[:, pl.ds(0, 128)], ring.at[h2, s], sems[h2]
                ).wait()

        def extract(iva, ivb, t, h2, s):
            i0 = lane(iva, ivb, 12 * h2 + s)
            cvec = jnp.broadcast_to(i0 & 127, (16,))
            pvec = jnp.broadcast_to(t + 12 * h2 + s, (16,))
            v0 = plsc.load_gather(ring.at[h2, s], [row0, cvec])
            v1 = plsc.load_gather(ring.at[h2, s], [row1, cvec])
            plsc.store_scatter(stg, [row0, pvec], v0)
            plsc.store_scatter(stg, [row1, pvec], v1)

        def loads(t):
            return idx_v[pl.ds(t, 16)], idx_v[pl.ds(t + 8, 16)]

        iva0, ivb0 = loads(0)
        for h2 in range(2):
            for s in range(12):
                issue(iva0, ivb0, h2, s)

        @pl.loop(0, BW - 32, step=24)
        def _body(t):
            iva, ivb = loads(t)
            ivn_a, ivn_b = loads(t + 24)
            for h2 in range(2):
                drain(h2)
                for s in range(12):
                    extract(iva, ivb, t, h2, s)
                for s in range(12):
                    issue(ivn_a, ivn_b, h2, s)

        t_tail = BW - 32
        iva, ivb = loads(t_tail)
        for h2 in range(2):
            drain(h2)
            for s in range(12):
                extract(iva, ivb, t_tail, h2, s)
        ivl = idx_v[pl.ds(BW - 16, 16)]
        for s in range(8):
            i0 = ivl[8 + s]
            off = pl.multiple_of((i0 >> 7) << 7, 128)
            pltpu.async_copy(
                zt_hbm.at[:, pl.ds(off, 128)], ring.at[0, s], sems[0]
            )
        for s in range(8):
            pltpu.make_async_copy(
                zt_hbm.at[:, pl.ds(0, 128)], ring.at[0, s], sems[0]
            ).wait()
        for s in range(8):
            i0 = ivl[8 + s]
            cvec = jnp.broadcast_to(i0 & 127, (16,))
            pvec = jnp.broadcast_to(BW - 8 + s, (16,))
            v0 = plsc.load_gather(ring.at[0, s], [row0, cvec])
            v1 = plsc.load_gather(ring.at[0, s], [row1, cvec])
            plsc.store_scatter(stg, [row0, pvec], v0)
            plsc.store_scatter(stg, [row1, pvec], v1)

        pltpu.sync_copy(stg, out_hbm.at[:, pl.ds(base, BW)])

    return gather_kernel(indices, Zt).T
